# Initial kernel scaffold; baseline (speedup 1.0000x reference)
import jax
import jax.numpy as jnp
from jax.experimental import pallas as pl


def kernel(customer_name, ticket_subject, customer_table, ticket_table):
    raise NotImplementedError("write your pallas kernel here")


# speedup vs baseline: 20.0570x; 20.0570x over previous
import jax
import jax.numpy as jnp
from jax import lax
from jax.experimental import pallas as pl
from jax.experimental.pallas import tpu as pltpu
from jax.experimental.pallas import tpu_sc as plsc

B = 16384
L = 50
D = 32
NC = 2
NS = 16
NW = NC * NS
PER_W = B // NW
C = 32
NCHUNK = PER_W // C
INV_L = 1.0 / L


def _body(cname_hbm, tsubj_hbm, ctab_hbm, ttab_hbm, out_hbm,
          cidx_v, crow_v, tidx_v, rows_v, outc_v, sem):
    wid = lax.axis_index("s") * NC + lax.axis_index("c")
    base = wid * PER_W

    pltpu.sync_copy(cname_hbm.at[pl.ds(base, PER_W)], cidx_v)
    pltpu.async_copy(ctab_hbm.at[cidx_v], crow_v, sem).wait()

    def chunk_body(k, _):
        row0 = base + k * C
        pltpu.sync_copy(tsubj_hbm.at[pl.ds(row0 * L, C * L)], tidx_v)
        pltpu.async_copy(ttab_hbm.at[tidx_v], rows_v, sem).wait()

        def elem_body(e, _):
            r0 = e * L
            a0 = rows_v[r0, pl.ds(0, 16)]
            a1 = rows_v[r0, pl.ds(16, 16)]
            for l in range(1, L):
                a0 = a0 + rows_v[r0 + l, pl.ds(0, 16)]
                a1 = a1 + rows_v[r0 + l, pl.ds(16, 16)]
            ce = k * C + e
            outc_v[e, pl.ds(0, 16)] = crow_v[ce, pl.ds(0, 16)]
            outc_v[e, pl.ds(16, 16)] = crow_v[ce, pl.ds(16, 16)]
            outc_v[e, pl.ds(32, 16)] = a0 * INV_L
            outc_v[e, pl.ds(48, 16)] = a1 * INV_L
            return 0

        lax.fori_loop(0, C, elem_body, 0)
        pltpu.sync_copy(outc_v, out_hbm.at[pl.ds(row0, C)])
        return 0

    lax.fori_loop(0, NCHUNK, chunk_body, 0)


@jax.jit
def kernel(customer_name, ticket_subject, customer_table, ticket_table):
    tsubj_flat = jnp.reshape(ticket_subject, (B * L,))
    mesh = plsc.VectorSubcoreMesh(core_axis_name="c", subcore_axis_name="s")
    k = pl.kernel(
        _body,
        out_type=jax.ShapeDtypeStruct((B, 2 * D), jnp.float32),
        mesh=mesh,
        scratch_types=[
            pltpu.VMEM((PER_W,), jnp.int32),
            pltpu.VMEM((PER_W, D), jnp.float32),
            pltpu.VMEM((C * L,), jnp.int32),
            pltpu.VMEM((C * L, D), jnp.float32),
            pltpu.VMEM((C, 2 * D), jnp.float32),
            pltpu.SemaphoreType.DMA,
        ],
        compiler_params=pltpu.CompilerParams(use_tc_tiling_on_sc=False),
    )
    return k(customer_name, tsubj_flat, customer_table, ticket_table)


# speedup vs baseline: 25.2555x; 1.2592x over previous
import jax
import jax.numpy as jnp
from jax import lax
from jax.experimental import pallas as pl
from jax.experimental.pallas import tpu as pltpu
from jax.experimental.pallas import tpu_sc as plsc

B = 16384
L = 50
D = 32
NC = 2
NS = 16
NW = NC * NS
PER_W = B // NW
C = 32
NCHUNK = PER_W // C
INV_L = 1.0 / L


def _body(cname_hbm, tsubj_hbm, ctab_hbm, ttab_hbm, out_hbm,
          cidx_v, crow_v, tidx0, tidx1, rows0, rows1, outc_v,
          csem, sem0, sem1):
    wid = lax.axis_index("s") * NC + lax.axis_index("c")
    base = wid * PER_W

    pltpu.sync_copy(cname_hbm.at[pl.ds(base, PER_W)], cidx_v)
    ccopy = pltpu.async_copy(ctab_hbm.at[cidx_v], crow_v, csem)

    def issue(k, tidx, rows, sem):
        pltpu.sync_copy(tsubj_hbm.at[pl.ds((base + k * C) * L, C * L)], tidx)
        return pltpu.async_copy(ttab_hbm.at[tidx], rows, sem)

    def reduce_chunk(k, rows):
        def elem_body(e, _):
            r0 = e * L
            a0 = rows[r0, pl.ds(0, 16)]
            a1 = rows[r0, pl.ds(16, 16)]
            for l in range(1, L):
                a0 = a0 + rows[r0 + l, pl.ds(0, 16)]
                a1 = a1 + rows[r0 + l, pl.ds(16, 16)]
            ce = k * C + e
            outc_v[e, pl.ds(0, 16)] = crow_v[ce, pl.ds(0, 16)]
            outc_v[e, pl.ds(16, 16)] = crow_v[ce, pl.ds(16, 16)]
            outc_v[e, pl.ds(32, 16)] = a0 * INV_L
            outc_v[e, pl.ds(48, 16)] = a1 * INV_L
            return 0

        lax.fori_loop(0, C, elem_body, 0)
        pltpu.sync_copy(outc_v, out_hbm.at[pl.ds(base + k * C, C)])

    issue(0, tidx0, rows0, sem0)
    ccopy.wait()

    def pair_body(p, _):
        ka = 2 * p
        issue(ka + 1, tidx1, rows1, sem1)
        pltpu.make_async_copy(ttab_hbm.at[tidx0], rows0, sem0).wait()
        reduce_chunk(ka, rows0)

        @pl.when(p < NCHUNK // 2 - 1)
        def _():
            issue(ka + 2, tidx0, rows0, sem0)

        pltpu.make_async_copy(ttab_hbm.at[tidx1], rows1, sem1).wait()
        reduce_chunk(ka + 1, rows1)
        return 0

    lax.fori_loop(0, NCHUNK // 2, pair_body, 0)


@jax.jit
def kernel(customer_name, ticket_subject, customer_table, ticket_table):
    tsubj_flat = jnp.reshape(ticket_subject, (B * L,))
    mesh = plsc.VectorSubcoreMesh(core_axis_name="c", subcore_axis_name="s")
    k = pl.kernel(
        _body,
        out_type=jax.ShapeDtypeStruct((B, 2 * D), jnp.float32),
        mesh=mesh,
        scratch_types=[
            pltpu.VMEM((PER_W,), jnp.int32),
            pltpu.VMEM((PER_W, D), jnp.float32),
            pltpu.VMEM((C * L,), jnp.int32),
            pltpu.VMEM((C * L,), jnp.int32),
            pltpu.VMEM((C * L, D), jnp.float32),
            pltpu.VMEM((C * L, D), jnp.float32),
            pltpu.VMEM((C, 2 * D), jnp.float32),
            pltpu.SemaphoreType.DMA,
            pltpu.SemaphoreType.DMA,
            pltpu.SemaphoreType.DMA,
        ],
        compiler_params=pltpu.CompilerParams(use_tc_tiling_on_sc=False),
    )
    return k(customer_name, tsubj_flat, customer_table, ticket_table)
